# Initial kernel scaffold; baseline (speedup 1.0000x reference)
import jax
import jax.numpy as jnp
from jax.experimental import pallas as pl


def kernel(inputs, table, W, b):
    raise NotImplementedError("write your pallas kernel here")


# speedup vs baseline: 6.0516x; 6.0516x over previous
import numpy as np
import jax
import jax.numpy as jnp
from jax import lax
from jax.experimental import pallas as pl
from jax.experimental.pallas import tpu as pltpu
from jax.experimental.pallas import tpu_sc as plsc

_B = 4096
_H = 50
_D = 128
_NCLS = 1000
_NCLS_PAD = 1024

_NC, _NS = 2, 16
_NW = _NC * _NS
_EPT = _B // _NW
_RPT = _EPT * _H
_RCH = 128
_NSTEP = _RPT // _RCH


def _pool_body(idx_hbm, sidx_hbm, table_hbm, out_hbm,
               idx_v, sidx_v, rows_v, acc_sh, sem):
    c = lax.axis_index("c")
    s = lax.axis_index("s")
    pltpu.sync_copy(idx_hbm.at[c, s], idx_v)
    pltpu.sync_copy(sidx_hbm.at[s], sidx_v)

    def zrow(i, _):
        zero = jnp.zeros((16,), jnp.float32)
        for cc in range(_D // 16):
            rows_v[i, cc * 16:(cc + 1) * 16] = zero
        return _
    lax.fori_loop(0, _RCH, zrow, None)
    pltpu.sync_copy(rows_v, acc_sh.at[pl.ds(s * _EPT, _EPT)])

    def step(t, _):
        pltpu.async_copy(table_hbm.at[idx_v.at[t]], rows_v, sem).wait()
        pltpu.sync_copy(rows_v, acc_sh.at[sidx_v.at[t]], add=True)
        return _
    lax.fori_loop(0, _NSTEP, step, None)

    pltpu.sync_copy(acc_sh.at[pl.ds(s * _EPT, _EPT)], rows_v)
    pltpu.sync_copy(rows_v, out_hbm.at[pl.ds((c * _NS + s) * _EPT, _EPT)])


def _pooled_sum(idx, table):
    mesh = plsc.VectorSubcoreMesh(core_axis_name="c", subcore_axis_name="s")
    r = np.arange(_RPT, dtype=np.int32) // _H
    sidx = (np.arange(_NS, dtype=np.int32)[:, None] * _EPT
            + r[None, :]).reshape(_NS, _NSTEP, _RCH)
    kfn = pl.kernel(
        _pool_body,
        out_type=jax.ShapeDtypeStruct((_B, _D), jnp.float32),
        mesh=mesh,
        scratch_types=[
            pltpu.VMEM((_NSTEP, _RCH), jnp.int32),
            pltpu.VMEM((_NSTEP, _RCH), jnp.int32),
            pltpu.VMEM((_RCH, _D), jnp.float32),
            pltpu.VMEM_SHARED((_NS * _EPT, _D), jnp.float32),
            pltpu.SemaphoreType.DMA,
        ],
    )
    return kfn(idx, jnp.asarray(sidx), table)


def _head_body(x_ref, w_ref, b_ref, o_ref):
    x = x_ref[...] * (1.0 / _H)
    o_ref[...] = lax.dot_general(
        x, w_ref[...], (((1,), (1,)), ((), ())),
        preferred_element_type=jnp.float32) + b_ref[...]


def _head(pooled_sum, wp, bp):
    m_blk = 1024
    return pl.pallas_call(
        _head_body,
        grid=(_B // m_blk,),
        in_specs=[
            pl.BlockSpec((m_blk, _D), lambda i: (i, 0)),
            pl.BlockSpec((_NCLS_PAD, _D), lambda i: (0, 0)),
            pl.BlockSpec((1, _NCLS_PAD), lambda i: (0, 0)),
        ],
        out_specs=pl.BlockSpec((m_blk, _NCLS_PAD), lambda i: (i, 0)),
        out_shape=jax.ShapeDtypeStruct((_B, _NCLS_PAD), jnp.float32),
    )(pooled_sum, wp, bp)


def kernel(inputs, table, W, b):
    idx = inputs.astype(jnp.int32).reshape(_NC, _NS, _NSTEP, _RCH)
    pooled = _pooled_sum(idx, table)
    wp = jnp.pad(W, ((0, _NCLS_PAD - _NCLS), (0, 0)))
    bp = jnp.pad(b, (0, _NCLS_PAD - _NCLS)).reshape(1, _NCLS_PAD)
    out = _head(pooled, wp, bp)
    return out[:, :_NCLS]


# speedup vs baseline: 7.4190x; 1.2260x over previous
import numpy as np
import jax
import jax.numpy as jnp
from jax import lax
from jax.experimental import pallas as pl
from jax.experimental.pallas import tpu as pltpu
from jax.experimental.pallas import tpu_sc as plsc

_B = 4096
_H = 50
_D = 128
_NCLS = 1000
_NCLS_PAD = 1024

_NC, _NS = 2, 16
_NW = _NC * _NS
_EPT = _B // _NW
_RPT = _EPT * _H
_RCH = 128
_NSTEP = _RPT // _RCH


def _pool_body(idx_hbm, sidx_hbm, table_hbm, out_hbm,
               idx_v, sidx_v, rows_a, rows_b, acc_sh, sem):
    c = lax.axis_index("c")
    s = lax.axis_index("s")
    pltpu.sync_copy(idx_hbm.at[c, s], idx_v)
    pltpu.sync_copy(sidx_hbm.at[s], sidx_v)

    def zrow(i, _):
        zero = jnp.zeros((16,), jnp.float32)
        for cc in range(_D // 16):
            rows_a[i, cc * 16:(cc + 1) * 16] = zero
        return _
    lax.fori_loop(0, _RCH, zrow, None)
    pltpu.sync_copy(rows_a, acc_sh.at[pl.ds(s * _EPT, _EPT)])

    def fire(t, buf):
        pltpu.async_copy(table_hbm.at[idx_v.at[t]], buf, sem)

    def drain(t, buf):
        pltpu.make_async_copy(table_hbm.at[idx_v.at[t]], buf, sem).wait()

    def scat(t, buf):
        pltpu.sync_copy(buf, acc_sh.at[sidx_v.at[t]], add=True)

    fire(0, rows_a)

    def step2(i, _):
        t = 2 * i
        drain(t, rows_a)
        fire(t + 1, rows_b)
        scat(t, rows_a)
        drain(t + 1, rows_b)

        @pl.when(i + 1 < _NSTEP // 2)
        def _():
            fire(t + 2, rows_a)
        scat(t + 1, rows_b)
        return _
    lax.fori_loop(0, _NSTEP // 2, step2, None)

    pltpu.sync_copy(acc_sh.at[pl.ds(s * _EPT, _EPT)], rows_a)
    pltpu.sync_copy(rows_a, out_hbm.at[pl.ds((c * _NS + s) * _EPT, _EPT)])


def _pooled_sum(idx, table):
    mesh = plsc.VectorSubcoreMesh(core_axis_name="c", subcore_axis_name="s")
    r = np.arange(_RPT, dtype=np.int32) // _H
    sidx = (np.arange(_NS, dtype=np.int32)[:, None] * _EPT
            + r[None, :]).reshape(_NS, _NSTEP, _RCH)
    kfn = pl.kernel(
        _pool_body,
        out_type=jax.ShapeDtypeStruct((_B, _D), jnp.float32),
        mesh=mesh,
        scratch_types=[
            pltpu.VMEM((_NSTEP, _RCH), jnp.int32),
            pltpu.VMEM((_NSTEP, _RCH), jnp.int32),
            pltpu.VMEM((_RCH, _D), jnp.float32),
            pltpu.VMEM((_RCH, _D), jnp.float32),
            pltpu.VMEM_SHARED((_NS * _EPT, _D), jnp.float32),
            pltpu.SemaphoreType.DMA,
        ],
    )
    return kfn(idx, jnp.asarray(sidx), table)


def _head_body(x_ref, w_ref, b_ref, o_ref):
    x = x_ref[...] * (1.0 / _H)
    o_ref[...] = lax.dot_general(
        x, w_ref[...], (((1,), (1,)), ((), ())),
        preferred_element_type=jnp.float32) + b_ref[...]


def _head(pooled_sum, wp, bp):
    m_blk = 1024
    return pl.pallas_call(
        _head_body,
        grid=(_B // m_blk,),
        in_specs=[
            pl.BlockSpec((m_blk, _D), lambda i: (i, 0)),
            pl.BlockSpec((_NCLS_PAD, _D), lambda i: (0, 0)),
            pl.BlockSpec((1, _NCLS_PAD), lambda i: (0, 0)),
        ],
        out_specs=pl.BlockSpec((m_blk, _NCLS_PAD), lambda i: (i, 0)),
        out_shape=jax.ShapeDtypeStruct((_B, _NCLS_PAD), jnp.float32),
    )(pooled_sum, wp, bp)


def kernel(inputs, table, W, b):
    idx = inputs.astype(jnp.int32).reshape(_NC, _NS, _NSTEP, _RCH)
    pooled = _pooled_sum(idx, table)
    wp = jnp.pad(W, ((0, _NCLS_PAD - _NCLS), (0, 0)))
    bp = jnp.pad(b, (0, _NCLS_PAD - _NCLS)).reshape(1, _NCLS_PAD)
    out = _head(pooled, wp, bp)
    return out[:, :_NCLS]


# speedup vs baseline: 9.0809x; 1.2240x over previous
import numpy as np
import jax
import jax.numpy as jnp
from jax import lax
from jax.experimental import pallas as pl
from jax.experimental.pallas import tpu as pltpu
from jax.experimental.pallas import tpu_sc as plsc

_B = 4096
_H = 50
_D = 128
_NCLS = 1000
_NCLS_PAD = 1024

_NC, _NS = 2, 16
_NW = _NC * _NS
_EPT = _B // _NW
_RPT = _EPT * _H
_ECH = 2
_RCH = _ECH * _H
_NSTEP = _EPT // _ECH
_CG = _D // 16


def _pool_body(idx_hbm, table_hbm, out_hbm, idx_v, rows_a, rows_b,
               out_v, sem_a, sem_b):
    c = lax.axis_index("c")
    s = lax.axis_index("s")
    pltpu.sync_copy(idx_hbm.at[c, s], idx_v)

    def fire(t, buf, sem):
        pltpu.async_copy(table_hbm.at[idx_v.at[t]], buf, sem)

    def drain(t, buf, sem):
        pltpu.make_async_copy(table_hbm.at[idx_v.at[t]], buf, sem).wait()

    def reduce_chunk(t, buf):
        for e in range(_ECH):
            def rowadd(r, acc):
                base = e * _H
                return tuple(
                    acc[cc] + buf[base + r, cc * 16:(cc + 1) * 16]
                    for cc in range(_CG))
            acc = lax.fori_loop(
                0, _H, rowadd,
                tuple(jnp.zeros((16,), jnp.float32) for _ in range(_CG)))
            row = t * _ECH + e
            for cc in range(_CG):
                out_v[row, cc * 16:(cc + 1) * 16] = acc[cc]

    fire(0, rows_a, sem_a)
    fire(1, rows_b, sem_b)

    def step2(i, _):
        t = 2 * i
        drain(t, rows_a, sem_a)
        reduce_chunk(t, rows_a)

        @pl.when(t + 2 < _NSTEP)
        def _():
            fire(t + 2, rows_a, sem_a)
        drain(t + 1, rows_b, sem_b)
        reduce_chunk(t + 1, rows_b)

        @pl.when(t + 3 < _NSTEP)
        def _():
            fire(t + 3, rows_b, sem_b)
        return _
    lax.fori_loop(0, _NSTEP // 2, step2, None)

    pltpu.sync_copy(out_v, out_hbm.at[pl.ds((c * _NS + s) * _EPT, _EPT)])


def _pooled_sum(idx, table):
    mesh = plsc.VectorSubcoreMesh(core_axis_name="c", subcore_axis_name="s")
    kfn = pl.kernel(
        _pool_body,
        out_type=jax.ShapeDtypeStruct((_B, _D), jnp.float32),
        mesh=mesh,
        scratch_types=[
            pltpu.VMEM((_NSTEP, _RCH), jnp.int32),
            pltpu.VMEM((_RCH, _D), jnp.float32),
            pltpu.VMEM((_RCH, _D), jnp.float32),
            pltpu.VMEM((_EPT, _D), jnp.float32),
            pltpu.SemaphoreType.DMA,
            pltpu.SemaphoreType.DMA,
        ],
    )
    return kfn(idx, table)


def _head_body(x_ref, w_ref, b_ref, o_ref):
    x = x_ref[...] * (1.0 / _H)
    o_ref[...] = lax.dot_general(
        x, w_ref[...], (((1,), (1,)), ((), ())),
        preferred_element_type=jnp.float32) + b_ref[...]


def _head(pooled_sum, wp, bp):
    m_blk = 1024
    return pl.pallas_call(
        _head_body,
        grid=(_B // m_blk,),
        in_specs=[
            pl.BlockSpec((m_blk, _D), lambda i: (i, 0)),
            pl.BlockSpec((_NCLS_PAD, _D), lambda i: (0, 0)),
            pl.BlockSpec((1, _NCLS_PAD), lambda i: (0, 0)),
        ],
        out_specs=pl.BlockSpec((m_blk, _NCLS_PAD), lambda i: (i, 0)),
        out_shape=jax.ShapeDtypeStruct((_B, _NCLS_PAD), jnp.float32),
    )(pooled_sum, wp, bp)


def kernel(inputs, table, W, b):
    idx = inputs.astype(jnp.int32).reshape(_NC, _NS, _NSTEP, _RCH)
    pooled = _pooled_sum(idx, table)
    wp = jnp.pad(W, ((0, _NCLS_PAD - _NCLS), (0, 0)))
    bp = jnp.pad(b, (0, _NCLS_PAD - _NCLS)).reshape(1, _NCLS_PAD)
    out = _head(pooled, wp, bp)
    return out[:, :_NCLS]


# speedup vs baseline: 10.9960x; 1.2109x over previous
import numpy as np
import jax
import jax.numpy as jnp
from jax import lax
from jax.experimental import pallas as pl
from jax.experimental.pallas import tpu as pltpu
from jax.experimental.pallas import tpu_sc as plsc

_B = 4096
_H = 50
_D = 128
_NCLS = 1000
_NCLS_PAD = 1024

_NC, _NS = 2, 16
_NW = _NC * _NS
_EPT = _B // _NW
_RPT = _EPT * _H
_ECH = 2
_RCH = _ECH * _H
_NSTEP = _EPT // _ECH
_CG = _D // 16


_NBUF = 4
_UNROLL = 5


def _pool_body(idx_hbm, table_hbm, out_hbm, idx_v,
               rows_0, rows_1, rows_2, rows_3,
               out_v, sem_0, sem_1, sem_2, sem_3):
    c = lax.axis_index("c")
    s = lax.axis_index("s")
    bufs = (rows_0, rows_1, rows_2, rows_3)
    sems = (sem_0, sem_1, sem_2, sem_3)
    pltpu.sync_copy(idx_hbm.at[c, s], idx_v)

    def fire(t, k):
        pltpu.async_copy(table_hbm.at[idx_v.at[t]], bufs[k], sems[k])

    def drain(t, k):
        pltpu.make_async_copy(table_hbm.at[idx_v.at[t]], bufs[k],
                              sems[k]).wait()

    def reduce_chunk(t, buf):
        for e in range(_ECH):
            def rowadd(j, acc):
                base = e * _H + j * _UNROLL
                for u in range(_UNROLL):
                    acc = tuple(
                        acc[cc] + buf[base + u, cc * 16:(cc + 1) * 16]
                        for cc in range(_CG))
                return acc
            acc = lax.fori_loop(
                0, _H // _UNROLL, rowadd,
                tuple(jnp.zeros((16,), jnp.float32) for _ in range(_CG)))
            row = t * _ECH + e
            for cc in range(_CG):
                out_v[row, cc * 16:(cc + 1) * 16] = acc[cc]

    for k in range(_NBUF - 1):
        fire(k, k)

    def step4(i, _):
        t0 = _NBUF * i
        for k in range(_NBUF):
            t = t0 + k
            drain(t, k)

            @pl.when(t + _NBUF - 1 < _NSTEP)
            def _():
                fire(t + _NBUF - 1, (k + _NBUF - 1) % _NBUF)
            reduce_chunk(t, bufs[k])
        return _
    lax.fori_loop(0, _NSTEP // _NBUF, step4, None)

    pltpu.sync_copy(out_v, out_hbm.at[pl.ds((c * _NS + s) * _EPT, _EPT)])


def _pooled_sum(idx, table):
    mesh = plsc.VectorSubcoreMesh(core_axis_name="c", subcore_axis_name="s")
    kfn = pl.kernel(
        _pool_body,
        out_type=jax.ShapeDtypeStruct((_B, _D), jnp.float32),
        mesh=mesh,
        scratch_types=(
            [pltpu.VMEM((_NSTEP, _RCH), jnp.int32)]
            + [pltpu.VMEM((_RCH, _D), jnp.float32)
               for _ in range(_NBUF)]
            + [pltpu.VMEM((_EPT, _D), jnp.float32)]
            + [pltpu.SemaphoreType.DMA for _ in range(_NBUF)]
        ),
    )
    return kfn(idx, table)


def _head_body(x_ref, w_ref, b_ref, o_ref):
    x = x_ref[...] * (1.0 / _H)
    o_ref[...] = lax.dot_general(
        x, w_ref[...], (((1,), (1,)), ((), ())),
        preferred_element_type=jnp.float32) + b_ref[...]


def _head(pooled_sum, w, b2):
    m_blk = 1024
    return pl.pallas_call(
        _head_body,
        grid=(_B // m_blk,),
        in_specs=[
            pl.BlockSpec((m_blk, _D), lambda i: (i, 0)),
            pl.BlockSpec((_NCLS, _D), lambda i: (0, 0)),
            pl.BlockSpec((1, _NCLS), lambda i: (0, 0)),
        ],
        out_specs=pl.BlockSpec((m_blk, _NCLS), lambda i: (i, 0)),
        out_shape=jax.ShapeDtypeStruct((_B, _NCLS), jnp.float32),
    )(pooled_sum, w, b2)


def kernel(inputs, table, W, b):
    idx = inputs.astype(jnp.int32).reshape(_NC, _NS, _NSTEP, _RCH)
    pooled = _pooled_sum(idx, table)
    return _head(pooled, W, b.reshape(1, _NCLS))
